# baseline (speedup 1.0000x reference)
import jax
import jax.numpy as jnp
from jax.experimental import pallas as pl
from jax.experimental.pallas import tpu as pltpu

_B = 8192
_XD = 2048
_HD = 1024
_ZD = 64
_K = 1024
_D = 64
_BETA = 0.25
_TB = 256
_GRID = _B // _TB


def _vqvae_body(x_ref, w1_ref, b1_ref, w2_ref, b2_ref, wp_ref, bp_ref,
                cb_ref, wd1_ref, bd1_ref, wd2_ref, bd2_ref,
                out_ref, vq_ref, pp_ref,
                counts_ref, sse_ref):
    i = pl.program_id(0)

    h = jnp.maximum(
        jnp.dot(x_ref[...], w1_ref[...], preferred_element_type=jnp.float32)
        + b1_ref[...], 0.0)
    z = jnp.dot(h, w2_ref[...], preferred_element_type=jnp.float32) + b2_ref[...]
    z_e = jnp.dot(z, wp_ref[...], preferred_element_type=jnp.float32) + bp_ref[...]

    cb = cb_ref[...]
    ones_row = jnp.ones((1, _D), dtype=jnp.float32)
    cb2 = jax.lax.dot_general(ones_row, cb * cb, (((1,), (1,)), ((), ())),
                              preferred_element_type=jnp.float32)
    t = jax.lax.dot_general(z_e, cb, (((1,), (1,)), ((), ())),
                            preferred_element_type=jnp.float32)
    z2 = jnp.sum(z_e * z_e, axis=1, keepdims=True)
    dist = (z2 + cb2) - 2.0 * t
    dmin = jnp.min(dist, axis=1, keepdims=True)
    kiota = jax.lax.broadcasted_iota(jnp.int32, (_TB, _K), 1)
    idx = jnp.min(jnp.where(dist == dmin, kiota, _K), axis=1, keepdims=True)
    one_hot = (kiota == idx).astype(jnp.float32)
    q = jnp.dot(one_hot, cb, preferred_element_type=jnp.float32)

    diff = q - z_e
    tile_sse = jnp.sum(diff * diff)
    tile_counts = jnp.sum(one_hot, axis=0, keepdims=True)

    @pl.when(i == 0)
    def _init():
        sse_ref[0] = 0.0
        counts_ref[...] = jnp.zeros_like(counts_ref)

    sse_ref[0] += tile_sse
    counts_ref[...] += tile_counts

    hd = jnp.maximum(
        jnp.dot(q.astype(jnp.bfloat16), wd1_ref[...],
                preferred_element_type=jnp.float32) + bd1_ref[...], 0.0)
    out_ref[...] = (
        jnp.dot(hd.astype(jnp.bfloat16), wd2_ref[...],
                preferred_element_type=jnp.float32) + bd2_ref[...])

    @pl.when(i == _GRID - 1)
    def _fin():
        e = sse_ref[0] / float(_B * _D)
        vq_ref[0, 0] = e + _BETA * e
        avg = counts_ref[...] * (1.0 / _B)
        pp_ref[0, 0] = jnp.exp(-jnp.sum(avg * jnp.log(avg + 1e-10)))


def kernel(x, W_enc1, b_enc1, W_enc2, b_enc2, W_pre, b_pre, codebook,
           W_dec1, b_dec1, W_dec2, b_dec2):
    full = lambda shape: pl.BlockSpec(shape, lambda i: (0,) * len(shape))
    out_shapes = (
        jax.ShapeDtypeStruct((_B, _XD), jnp.float32),
        jax.ShapeDtypeStruct((1, 1), jnp.float32),
        jax.ShapeDtypeStruct((1, 1), jnp.float32),
    )
    x_recon, vq, pp = pl.pallas_call(
        _vqvae_body,
        grid=(_GRID,),
        in_specs=[
            pl.BlockSpec((_TB, _XD), lambda i: (i, 0)),
            full((_XD, _HD)), full((1, _HD)),
            full((_HD, _ZD)), full((1, _ZD)),
            full((_ZD, _D)), full((1, _D)),
            full((_K, _D)),
            full((_D, _HD)), full((1, _HD)),
            full((_HD, _XD)), full((1, _XD)),
        ],
        out_specs=(
            pl.BlockSpec((_TB, _XD), lambda i: (i, 0)),
            pl.BlockSpec(memory_space=pltpu.SMEM),
            pl.BlockSpec(memory_space=pltpu.SMEM),
        ),
        out_shape=out_shapes,
        scratch_shapes=[
            pltpu.VMEM((1, _K), jnp.float32),
            pltpu.SMEM((1,), jnp.float32),
        ],
        compiler_params=pltpu.CompilerParams(
            dimension_semantics=("arbitrary",),
        ),
    )(
        x,
        W_enc1, b_enc1.reshape(1, _HD),
        W_enc2, b_enc2.reshape(1, _ZD),
        W_pre, b_pre.reshape(1, _D),
        codebook,
        W_dec1.astype(jnp.bfloat16), b_dec1.reshape(1, _HD),
        W_dec2.astype(jnp.bfloat16), b_dec2.reshape(1, _XD),
    )
    return x_recon, vq[0, 0], pp[0, 0]


# speedup vs baseline: 1.1635x; 1.1635x over previous
import jax
import jax.numpy as jnp
from jax.experimental import pallas as pl
from jax.experimental.pallas import tpu as pltpu

_B = 8192
_XD = 2048
_HD = 1024
_ZD = 64
_K = 1024
_D = 64
_BETA = 0.25
_TB = 1024
_GRID = _B // _TB


def _vqvae_body(x_ref, w1_ref, b1_ref, w2_ref, b2_ref, wp_ref, bp_ref,
                cb_ref, wd1_ref, bd1_ref, wd2_ref, bd2_ref,
                out_ref, vq_ref, pp_ref,
                counts_ref, sse_ref):
    i = pl.program_id(0)

    h = jnp.maximum(
        jnp.dot(x_ref[...], w1_ref[...], preferred_element_type=jnp.float32)
        + b1_ref[...], 0.0)
    z = jnp.dot(h, w2_ref[...], preferred_element_type=jnp.float32) + b2_ref[...]
    z_e = jnp.dot(z, wp_ref[...], preferred_element_type=jnp.float32) + bp_ref[...]

    cb = cb_ref[...]
    ones_row = jnp.ones((1, _D), dtype=jnp.float32)
    cb2 = jax.lax.dot_general(ones_row, cb * cb, (((1,), (1,)), ((), ())),
                              preferred_element_type=jnp.float32)
    t = jax.lax.dot_general(z_e, cb, (((1,), (1,)), ((), ())),
                            preferred_element_type=jnp.float32)
    z2 = jnp.sum(z_e * z_e, axis=1, keepdims=True)
    dist = (z2 + cb2) - 2.0 * t
    dmin = jnp.min(dist, axis=1, keepdims=True)
    kiota = jax.lax.broadcasted_iota(jnp.int32, (_TB, _K), 1)
    idx = jnp.min(jnp.where(dist == dmin, kiota, _K), axis=1, keepdims=True)
    one_hot = (kiota == idx).astype(jnp.float32)
    q = jnp.dot(one_hot, cb, preferred_element_type=jnp.float32)

    diff = q - z_e
    tile_sse = jnp.sum(diff * diff)
    tile_counts = jnp.sum(one_hot, axis=0, keepdims=True)

    @pl.when(i == 0)
    def _init():
        sse_ref[0] = 0.0
        counts_ref[...] = jnp.zeros_like(counts_ref)

    sse_ref[0] += tile_sse
    counts_ref[...] += tile_counts

    hd = jnp.maximum(
        jnp.dot(q.astype(jnp.bfloat16), wd1_ref[...],
                preferred_element_type=jnp.float32) + bd1_ref[...], 0.0)
    out_ref[...] = (
        jnp.dot(hd.astype(jnp.bfloat16), wd2_ref[...],
                preferred_element_type=jnp.float32) + bd2_ref[...])

    @pl.when(i == _GRID - 1)
    def _fin():
        e = sse_ref[0] / float(_B * _D)
        vq_ref[0, 0] = e + _BETA * e
        avg = counts_ref[...] * (1.0 / _B)
        pp_ref[0, 0] = jnp.exp(-jnp.sum(avg * jnp.log(avg + 1e-10)))


def kernel(x, W_enc1, b_enc1, W_enc2, b_enc2, W_pre, b_pre, codebook,
           W_dec1, b_dec1, W_dec2, b_dec2):
    full = lambda shape: pl.BlockSpec(shape, lambda i: (0,) * len(shape))
    out_shapes = (
        jax.ShapeDtypeStruct((_B, _XD), jnp.float32),
        jax.ShapeDtypeStruct((1, 1), jnp.float32),
        jax.ShapeDtypeStruct((1, 1), jnp.float32),
    )
    x_recon, vq, pp = pl.pallas_call(
        _vqvae_body,
        grid=(_GRID,),
        in_specs=[
            pl.BlockSpec((_TB, _XD), lambda i: (i, 0)),
            full((_XD, _HD)), full((1, _HD)),
            full((_HD, _ZD)), full((1, _ZD)),
            full((_ZD, _D)), full((1, _D)),
            full((_K, _D)),
            full((_D, _HD)), full((1, _HD)),
            full((_HD, _XD)), full((1, _XD)),
        ],
        out_specs=(
            pl.BlockSpec((_TB, _XD), lambda i: (i, 0)),
            pl.BlockSpec(memory_space=pltpu.SMEM),
            pl.BlockSpec(memory_space=pltpu.SMEM),
        ),
        out_shape=out_shapes,
        scratch_shapes=[
            pltpu.VMEM((1, _K), jnp.float32),
            pltpu.SMEM((1,), jnp.float32),
        ],
        compiler_params=pltpu.CompilerParams(
            dimension_semantics=("arbitrary",),
        ),
    )(
        x,
        W_enc1, b_enc1.reshape(1, _HD),
        W_enc2, b_enc2.reshape(1, _ZD),
        W_pre, b_pre.reshape(1, _D),
        codebook,
        W_dec1.astype(jnp.bfloat16), b_dec1.reshape(1, _HD),
        W_dec2.astype(jnp.bfloat16), b_dec2.reshape(1, _XD),
    )
    return x_recon, vq[0, 0], pp[0, 0]


# speedup vs baseline: 1.2192x; 1.0478x over previous
import jax
import jax.numpy as jnp
from jax.experimental import pallas as pl
from jax.experimental.pallas import tpu as pltpu

_B = 8192
_XD = 2048
_HD = 1024
_ZD = 64
_K = 1024
_D = 64
_BETA = 0.25
_TB = 1024
_GRID = _B // _TB


def _vqvae_body(x_ref, w1_ref, b1_ref, w2_ref, b2_ref, wp_ref, bp_ref,
                cb_ref, wd1_ref, bd1_ref, wd2_ref, bd2_ref,
                out_ref, vq_ref, pp_ref,
                counts_ref, sse_ref, wd1c_ref, wd2c_ref):
    i = pl.program_id(0)

    h = jnp.maximum(
        jnp.dot(x_ref[...], w1_ref[...], preferred_element_type=jnp.float32)
        + b1_ref[...], 0.0)
    z = jnp.dot(h, w2_ref[...], preferred_element_type=jnp.float32) + b2_ref[...]
    z_e = jnp.dot(z, wp_ref[...], preferred_element_type=jnp.float32) + bp_ref[...]

    cb = cb_ref[...]
    ones_row = jnp.ones((1, _D), dtype=jnp.float32)
    cb2 = jax.lax.dot_general(ones_row, cb * cb, (((1,), (1,)), ((), ())),
                              preferred_element_type=jnp.float32)
    t = jax.lax.dot_general(z_e, cb, (((1,), (1,)), ((), ())),
                            preferred_element_type=jnp.float32)
    z2 = jnp.sum(z_e * z_e, axis=1, keepdims=True)
    dist = (z2 + cb2) - 2.0 * t
    dmin = jnp.min(dist, axis=1, keepdims=True)
    kiota = jax.lax.broadcasted_iota(jnp.int32, (_TB, _K), 1)
    idx = jnp.min(jnp.where(dist == dmin, kiota, _K), axis=1, keepdims=True)
    one_hot = (kiota == idx).astype(jnp.float32)
    q = jnp.dot(one_hot, cb, preferred_element_type=jnp.float32)

    diff = q - z_e
    tile_sse = jnp.sum(diff * diff)
    tile_counts = jnp.sum(one_hot, axis=0, keepdims=True)

    @pl.when(i == 0)
    def _init():
        sse_ref[0] = 0.0
        counts_ref[...] = jnp.zeros_like(counts_ref)
        wd1c_ref[...] = wd1_ref[...].astype(jnp.bfloat16)
        wd2c_ref[...] = wd2_ref[...].astype(jnp.bfloat16)

    sse_ref[0] += tile_sse
    counts_ref[...] += tile_counts

    hd = jnp.maximum(
        jnp.dot(q.astype(jnp.bfloat16), wd1c_ref[...],
                preferred_element_type=jnp.float32) + bd1_ref[...], 0.0)
    out_ref[...] = (
        jnp.dot(hd.astype(jnp.bfloat16), wd2c_ref[...],
                preferred_element_type=jnp.float32) + bd2_ref[...])

    @pl.when(i == _GRID - 1)
    def _fin():
        e = sse_ref[0] / float(_B * _D)
        vq_ref[0, 0] = e + _BETA * e
        avg = counts_ref[...] * (1.0 / _B)
        pp_ref[0, 0] = jnp.exp(-jnp.sum(avg * jnp.log(avg + 1e-10)))


def kernel(x, W_enc1, b_enc1, W_enc2, b_enc2, W_pre, b_pre, codebook,
           W_dec1, b_dec1, W_dec2, b_dec2):
    full = lambda shape: pl.BlockSpec(shape, lambda i: (0,) * len(shape))
    out_shapes = (
        jax.ShapeDtypeStruct((_B, _XD), jnp.float32),
        jax.ShapeDtypeStruct((1, 1), jnp.float32),
        jax.ShapeDtypeStruct((1, 1), jnp.float32),
    )
    x_recon, vq, pp = pl.pallas_call(
        _vqvae_body,
        grid=(_GRID,),
        in_specs=[
            pl.BlockSpec((_TB, _XD), lambda i: (i, 0)),
            full((_XD, _HD)), full((1, _HD)),
            full((_HD, _ZD)), full((1, _ZD)),
            full((_ZD, _D)), full((1, _D)),
            full((_K, _D)),
            full((_D, _HD)), full((1, _HD)),
            full((_HD, _XD)), full((1, _XD)),
        ],
        out_specs=(
            pl.BlockSpec((_TB, _XD), lambda i: (i, 0)),
            pl.BlockSpec(memory_space=pltpu.SMEM),
            pl.BlockSpec(memory_space=pltpu.SMEM),
        ),
        out_shape=out_shapes,
        scratch_shapes=[
            pltpu.VMEM((1, _K), jnp.float32),
            pltpu.SMEM((1,), jnp.float32),
            pltpu.VMEM((_D, _HD), jnp.bfloat16),
            pltpu.VMEM((_HD, _XD), jnp.bfloat16),
        ],
        compiler_params=pltpu.CompilerParams(
            dimension_semantics=("arbitrary",),
            vmem_limit_bytes=100 * 1024 * 1024,
        ),
    )(
        x,
        W_enc1, b_enc1.reshape(1, _HD),
        W_enc2, b_enc2.reshape(1, _ZD),
        W_pre, b_pre.reshape(1, _D),
        codebook,
        W_dec1, b_dec1.reshape(1, _HD),
        W_dec2, b_dec2.reshape(1, _XD),
    )
    return x_recon, vq[0, 0], pp[0, 0]
